# baseline (speedup 1.0000x reference)
import jax
import jax.numpy as jnp
from jax import lax
from jax.experimental import pallas as pl
from jax.experimental.pallas import tpu as pltpu
from jax.experimental.pallas import tpu_sc as plsc

B, ROW, COL = 1024, 200, 128
NC, NS, L = 2, 16, 16
NW = NC * NS
KCOL = COL // L

NSC = 640
SC_PER_W = NSC // NW
NTC = B - NSC
G = 32
NCH = NTC // G


def _sc_body(tensor_hbm, rp_hbm, cp_hbm, out_hbm,
             in_v0, in_v1, out_v0, out_v1, rp_v, cp_v,
             sin0, sin1, sout0, sout1):
    wid = lax.axis_index("s") * NC + lax.axis_index("c")
    base_img = wid * SC_PER_W

    pltpu.sync_copy(rp_hbm, rp_v)
    pltpu.sync_copy(cp_hbm, cp_v)

    in_bufs, out_bufs = (in_v0, in_v1), (out_v0, out_v1)
    sins, souts = (sin0, sin1), (sout0, sout1)

    cps = [cp_v[k, :] for k in range(KCOL)]

    pltpu.async_copy(tensor_hbm.at[base_img], in_v0, sin0)

    def per_pair(p, _):
        for bslot in range(2):
            t = p * 2 + bslot
            in_b, out_b = in_bufs[bslot], out_bufs[bslot]
            s_in, s_out = sins[bslot], souts[bslot]

            @pl.when(t + 1 < SC_PER_W)
            def _():
                pltpu.async_copy(tensor_hbm.at[base_img + t + 1],
                                 in_bufs[1 - bslot], sins[1 - bslot])

            pltpu.make_async_copy(tensor_hbm.at[base_img + t], in_b,
                                  s_in).wait()

            @pl.when(t >= 2)
            def _():
                pltpu.make_async_copy(out_b, out_hbm.at[base_img + t - 2],
                                      s_out).wait()

            @plsc.parallel_loop(0, ROW, 1, unroll=4)
            def _(i):
                row_splat = rp_v[i, :]
                for k in range(KCOL):
                    x = plsc.load_gather(in_b, [row_splat, cps[k]])
                    out_b[i, pl.ds(k * L, L)] = x

            pltpu.async_copy(out_b, out_hbm.at[base_img + t], s_out)
        return 0

    lax.fori_loop(0, SC_PER_W // 2, per_pair, 0)

    pltpu.make_async_copy(out_v0, out_hbm.at[base_img + SC_PER_W - 2],
                          sout0).wait()
    pltpu.make_async_copy(out_v1, out_hbm.at[base_img + SC_PER_W - 1],
                          sout1).wait()


def _tc_grid_body(x_ref, rmat_ref, cmat_ref, o_ref):
    x = x_ref[...].reshape(G * ROW, COL)
    xc = lax.dot_general(x, cmat_ref[...], (((1,), (0,)), ((), ())),
                         preferred_element_type=jnp.float32)
    rmat = rmat_ref[...]
    for g in range(G):
        o_ref[g] = lax.dot_general(rmat, xc[g * ROW:(g + 1) * ROW, :],
                                   (((1,), (0,)), ((), ())),
                                   preferred_element_type=jnp.float32)


@jax.jit
def _permute(tensor, rp_bcast, cp_2d, rmat, cmat):
    sc_fn = pl.kernel(
        _sc_body,
        out_type=jax.ShapeDtypeStruct((NSC, ROW, COL), jnp.float32),
        mesh=plsc.VectorSubcoreMesh(core_axis_name="c", subcore_axis_name="s"),
        compiler_params=pltpu.CompilerParams(needs_layout_passes=False),
        scratch_types=[
            pltpu.VMEM((ROW, COL), jnp.float32),
            pltpu.VMEM((ROW, COL), jnp.float32),
            pltpu.VMEM((ROW, COL), jnp.float32),
            pltpu.VMEM((ROW, COL), jnp.float32),
            pltpu.VMEM((ROW, L), jnp.int32),
            pltpu.VMEM((KCOL, L), jnp.int32),
            pltpu.SemaphoreType.DMA,
            pltpu.SemaphoreType.DMA,
            pltpu.SemaphoreType.DMA,
            pltpu.SemaphoreType.DMA,
        ],
    )
    tc_fn = pl.pallas_call(
        _tc_grid_body,
        grid=(NCH,),
        in_specs=[
            pl.BlockSpec((G, ROW, COL), lambda c: (NSC // G + c, 0, 0)),
            pl.BlockSpec((ROW, ROW), lambda c: (0, 0)),
            pl.BlockSpec((COL, COL), lambda c: (0, 0)),
        ],
        out_specs=pl.BlockSpec((G, ROW, COL), lambda c: (c, 0, 0)),
        out_shape=jax.ShapeDtypeStruct((NTC, ROW, COL), jnp.float32),
    )
    out_sc = sc_fn(tensor, rp_bcast, cp_2d)
    out_tc = tc_fn(tensor, rmat, cmat)
    return jnp.concatenate([out_sc, out_tc], axis=0)


def kernel(tensor, rowperm, colperm):
    rp = rowperm.astype(jnp.int32)
    cp = colperm.astype(jnp.int32)
    rp_bcast = jnp.broadcast_to(rp[:, None], (ROW, L)).astype(jnp.int32)
    cp_2d = cp.reshape(KCOL, L)
    rmat = jax.nn.one_hot(rp, ROW, dtype=jnp.float32)
    cmat = jax.nn.one_hot(cp, COL, dtype=jnp.float32).T
    return _permute(tensor, rp_bcast, cp_2d, rmat, cmat)


# speedup vs baseline: 1.6621x; 1.6621x over previous
import jax
import jax.numpy as jnp
from jax import lax
from jax.experimental import pallas as pl
from jax.experimental.pallas import tpu as pltpu
from jax.experimental.pallas import tpu_sc as plsc

B, ROW, COL = 1024, 200, 128
NC, NS, L = 2, 16, 16
NW = NC * NS
IMGS_PER_W = B // NW
KCOL = COL // L


def _body(tensor_hbm, rp_hbm, cp_hbm, out_hbm,
          in_v0, in_v1, out_v0, out_v1, rp_v, cp_v,
          sin0, sin1, sout0, sout1):
    wid = lax.axis_index("s") * NC + lax.axis_index("c")
    base_img = wid * IMGS_PER_W

    pltpu.sync_copy(rp_hbm, rp_v)
    pltpu.sync_copy(cp_hbm, cp_v)

    in_bufs, out_bufs = (in_v0, in_v1), (out_v0, out_v1)
    sins, souts = (sin0, sin1), (sout0, sout1)

    cps = [cp_v[pl.ds(k * L, L)] for k in range(KCOL)]

    pltpu.async_copy(tensor_hbm.at[base_img], in_v0, sin0)

    def per_pair(p, _):
        for bslot in range(2):
            t = p * 2 + bslot
            in_b, out_b = in_bufs[bslot], out_bufs[bslot]
            s_in, s_out = sins[bslot], souts[bslot]

            @pl.when(t + 1 < IMGS_PER_W)
            def _():
                pltpu.async_copy(tensor_hbm.at[base_img + t + 1],
                                 in_bufs[1 - bslot], sins[1 - bslot])

            pltpu.make_async_copy(tensor_hbm.at[base_img + t], in_b,
                                  s_in).wait()

            @pl.when(t >= 2)
            def _():
                pltpu.make_async_copy(out_b, out_hbm.at[base_img + t - 2],
                                      s_out).wait()

            @plsc.parallel_loop(0, ROW, 1, unroll=4)
            def _(i):
                row_splat = plsc.load_gather(
                    rp_v, [jnp.full((L,), i, jnp.int32)])
                for k in range(KCOL):
                    x = plsc.load_gather(in_b, [row_splat, cps[k]])
                    out_b[i, pl.ds(k * L, L)] = x

            pltpu.async_copy(out_b, out_hbm.at[base_img + t], s_out)
        return 0

    lax.fori_loop(0, IMGS_PER_W // 2, per_pair, 0)

    pltpu.make_async_copy(out_v0, out_hbm.at[base_img + IMGS_PER_W - 2],
                          sout0).wait()
    pltpu.make_async_copy(out_v1, out_hbm.at[base_img + IMGS_PER_W - 1],
                          sout1).wait()


@jax.jit
def _permute(tensor, rowperm, colperm):
    kfn = pl.kernel(
        _body,
        out_type=jax.ShapeDtypeStruct((B, ROW, COL), jnp.float32),
        mesh=plsc.VectorSubcoreMesh(core_axis_name="c", subcore_axis_name="s"),
        compiler_params=pltpu.CompilerParams(needs_layout_passes=False),
        scratch_types=[
            pltpu.VMEM((ROW, COL), jnp.float32),
            pltpu.VMEM((ROW, COL), jnp.float32),
            pltpu.VMEM((ROW, COL), jnp.float32),
            pltpu.VMEM((ROW, COL), jnp.float32),
            pltpu.VMEM((ROW,), jnp.int32),
            pltpu.VMEM((COL,), jnp.int32),
            pltpu.SemaphoreType.DMA,
            pltpu.SemaphoreType.DMA,
            pltpu.SemaphoreType.DMA,
            pltpu.SemaphoreType.DMA,
        ],
    )
    return kfn(tensor, rowperm, colperm)


def kernel(tensor, rowperm, colperm):
    return _permute(tensor, rowperm.astype(jnp.int32),
                    colperm.astype(jnp.int32))
